# Initial kernel scaffold; baseline (speedup 1.0000x reference)
import functools

import jax
import jax.numpy as jnp
from jax import lax
from jax.experimental import pallas as pl
from jax.experimental.pallas import tpu as pltpu
from jax.experimental.pallas import tpu_sc as plsc

H = 128
MIDC = 512
NC, NS, LANES = 2, 16, 16
NW = NC * NS
CH = 128
TBL = 10016
ROW_BLK = 2000


def _dotc(a, b):
    return lax.dot_general(a, b, (((1,), (1,)), ((), ())),
                           preferred_element_type=jnp.float32)


def _layernorm(v, g, b):
    mu = jnp.mean(v, axis=-1, keepdims=True)
    var = jnp.mean((v - mu) ** 2, axis=-1, keepdims=True)
    return (v - mu) / jnp.sqrt(var + 1e-12) * g + b


def _gelu(v):
    return 0.5 * v * (1.0 + lax.erf(v * 0.7071067811865476))


def _tc_pre_body(x, wpre, bpre, w0, a_s, a_d, h0_o, hw_o, as_o, ad_o):
    h0 = jnp.maximum(_dotc(x[...], wpre[...]) + bpre[...], 0.0)
    hw = _dotc(h0, w0[...])
    h0_o[...] = h0
    hw_o[...] = hw
    as_o[...] = hw @ a_s[...]
    ad_o[...] = hw @ a_d[...]


def _post_block(num0, num1, den0, den1, prex, gatb, lng, lnb, wm, bm, wo, bo,
                rg, rb):
    agg = num0[0] + num1[0]
    den = den0[0][:, None] + den1[0][:, None] + 1e-16
    out = agg / den + gatb[...]
    h = _layernorm(prex[...] + _gelu(out), lng[...], lnb[...])
    mid = _gelu(_dotc(h, wm[...]) + bm[...])
    mid = _dotc(mid, wo[...]) + bo[...]
    return _layernorm(h + mid, rg[...], rb[...])


def _tc_post0_body(num0, num1, den0, den1, prex, gatb, lng, lnb, wm, bm, wo,
                   bo, rg, rb, w1, a_s, a_d, hw_o, as_o, ad_o):
    h2 = _post_block(num0, num1, den0, den1, prex, gatb, lng, lnb, wm, bm,
                     wo, bo, rg, rb)
    hw = _dotc(h2, w1[...])
    hw_o[...] = hw
    as_o[...] = hw @ a_s[...]
    ad_o[...] = hw @ a_d[...]


def _tc_post1_body(num0, num1, den0, den1, prex, gatb, lng, lnb, wm, bm, wo,
                   bo, rg, rb, wout, bout, y_o):
    h2 = _post_block(num0, num1, den0, den1, prex, gatb, lng, lnb, wm, bm,
                     wo, bo, rg, rb)
    y_o[...] = jnp.maximum(_dotc(h2, wout[...]) + bout[...], 0.0)


def _row_spec():
    return pl.BlockSpec((ROW_BLK, H), lambda i: (i, 0))


def _full(shape):
    return pl.BlockSpec(shape, lambda i: (0,) * len(shape))


def _sc_edge_body(nchunks, nacc, src_h, dst_h, hw_h, as_h, ad_h,
                  num_h, den_h, as_v, ad_v, srcv, dstv, rows, wbuf,
                  out_acc, den_acc, sem):
    cid = lax.axis_index("c")
    sid = lax.axis_index("s")
    wid = cid * NS + sid
    rows_per_tile = nacc // NS
    blocks_per_tile = rows_per_tile // CH

    pltpu.sync_copy(as_h, as_v)
    pltpu.sync_copy(ad_h, ad_v)

    z = jnp.zeros((LANES,), jnp.float32)

    def _zero_rows(j, _):
        for c in range(H // LANES):
            rows[j, pl.ds(c * LANES, LANES)] = z
        return 0

    lax.fori_loop(0, CH, _zero_rows, 0)
    for c in range(CH // LANES):
        wbuf[0, pl.ds(c * LANES, LANES)] = z
    for q in range(blocks_per_tile):
        r0 = sid * rows_per_tile + q * CH
        pltpu.sync_copy(rows, out_acc.at[pl.ds(r0, CH)])
        pltpu.sync_copy(wbuf.at[0], den_acc.at[pl.ds(r0, CH)])
    plsc.subcore_barrier()

    def _chunk(q, _):
        base = (wid * nchunks + q) * CH
        pltpu.sync_copy(src_h.at[pl.ds(base, CH)], srcv.at[0])
        pltpu.sync_copy(dst_h.at[pl.ds(base, CH)], dstv.at[0])
        cp = pltpu.async_copy(hw_h.at[srcv.at[0]], rows, sem)
        for i in range(CH // LANES):
            sv = srcv[0, pl.ds(i * LANES, LANES)]
            dv = dstv[0, pl.ds(i * LANES, LANES)]
            s = plsc.load_gather(as_v, [sv]) + plsc.load_gather(ad_v, [dv])
            e = jnp.where(s >= 0.0, s, 0.2 * s)
            wbuf[0, pl.ds(i * LANES, LANES)] = jnp.exp(e)
        cp.wait()

        def _scale(j, _):
            wj = wbuf[0, j]
            for c in range(H // LANES):
                sl = pl.ds(c * LANES, LANES)
                rows[j, sl] = rows[j, sl] * wj
            return 0

        lax.fori_loop(0, CH, _scale, 0)
        pltpu.sync_copy(rows, out_acc.at[dstv.at[0]], add=True)
        pltpu.sync_copy(wbuf.at[0], den_acc.at[dstv.at[0]], add=True)
        return 0

    lax.fori_loop(0, nchunks, _chunk, 0)
    plsc.subcore_barrier()

    for q in range(blocks_per_tile):
        r0 = sid * rows_per_tile + q * CH
        pltpu.sync_copy(out_acc.at[pl.ds(r0, CH)],
                        num_h.at[cid].at[pl.ds(r0, CH)])
        pltpu.sync_copy(den_acc.at[pl.ds(r0, CH)],
                        den_h.at[cid].at[pl.ds(r0, CH)])


def _sc_edge(src, dst, hw, as_pad, ad_pad, nchunks, nacc):
    mesh = plsc.VectorSubcoreMesh(core_axis_name="c", subcore_axis_name="s",
                                  num_cores=NC, num_subcores=NS)
    fn = pl.kernel(
        functools.partial(_sc_edge_body, nchunks, nacc),
        out_type=(jax.ShapeDtypeStruct((NC, nacc, H), jnp.float32),
                  jax.ShapeDtypeStruct((NC, nacc), jnp.float32)),
        mesh=mesh,
        scratch_types=(
            pltpu.VMEM((TBL,), jnp.float32),
            pltpu.VMEM((TBL,), jnp.float32),
            pltpu.VMEM((1, CH), jnp.int32),
            pltpu.VMEM((1, CH), jnp.int32),
            pltpu.VMEM((CH, H), jnp.float32),
            pltpu.VMEM((1, CH), jnp.float32),
            pltpu.VMEM_SHARED((nacc, H), jnp.float32),
            pltpu.VMEM_SHARED((nacc,), jnp.float32),
            pltpu.SemaphoreType.DMA,
        ),
    )
    return fn(src, dst, hw, as_pad, ad_pad)


def kernel(x, edge_index, W_pre, b_pre, gat_W0, gat_as0, gat_ad0, gat_b0,
           res_Wm0, res_bm0, res_Wo0, res_bo0, res_g0, res_bb0, gat_W1,
           gat_as1, gat_ad1, gat_b1, res_Wm1, res_bm1, res_Wo1, res_bo1,
           res_g1, res_bb1, ln_g, ln_b, W_out, b_out):
    n = x.shape[0]
    e = edge_index.shape[1]
    et = e + n
    nchunks = -(-et // (NW * CH))
    ep = NW * nchunks * CH
    nacc = -(-(n + LANES) // (NS * CH)) * NS * CH

    loops = jnp.arange(n, dtype=edge_index.dtype)
    src = jnp.concatenate([edge_index[0], loops,
                           jnp.zeros((ep - et,), edge_index.dtype)])
    dst = jnp.concatenate([edge_index[1], loops,
                           jnp.full((ep - et,), n, edge_index.dtype)])

    grid = n // ROW_BLK
    vec = pl.BlockSpec((H, 1), lambda i: (0, 0))
    row1 = pl.BlockSpec((ROW_BLK, 1), lambda i: (i, 0))
    nb = pl.BlockSpec((1, ROW_BLK, H), lambda i: (0, i, 0))
    nb1 = pl.BlockSpec((1, ROW_BLK, H), lambda i: (1, i, 0))
    db = pl.BlockSpec((1, ROW_BLK), lambda i: (0, i))
    db1 = pl.BlockSpec((1, ROW_BLK), lambda i: (1, i))

    h0, hw0, as0, ad0 = pl.pallas_call(
        _tc_pre_body,
        grid=(grid,),
        in_specs=[_row_spec(), _full((H, H)), _full((1, H)), _full((H, H)),
                  vec, vec],
        out_specs=[_row_spec(), _row_spec(), row1, row1],
        out_shape=[jax.ShapeDtypeStruct((n, H), jnp.float32),
                   jax.ShapeDtypeStruct((n, H), jnp.float32),
                   jax.ShapeDtypeStruct((n, 1), jnp.float32),
                   jax.ShapeDtypeStruct((n, 1), jnp.float32)],
    )(x, W_pre, b_pre.reshape(1, H), gat_W0, gat_as0.reshape(H, 1),
      gat_ad0.reshape(H, 1))

    pad = jnp.zeros((TBL - n,), jnp.float32)
    num0, den0 = _sc_edge(src, dst, hw0,
                          jnp.concatenate([as0.reshape(-1), pad]),
                          jnp.concatenate([ad0.reshape(-1), pad]),
                          nchunks, nacc)

    hw1, as1, ad1 = pl.pallas_call(
        _tc_post0_body,
        grid=(grid,),
        in_specs=[nb, nb1, db, db1, _row_spec(), _full((1, H)), _full((1, H)),
                  _full((1, H)), _full((MIDC, H)), _full((1, MIDC)),
                  _full((H, MIDC)), _full((1, H)), _full((1, H)),
                  _full((1, H)), _full((H, H)), vec, vec],
        out_specs=[_row_spec(), row1, row1],
        out_shape=[jax.ShapeDtypeStruct((n, H), jnp.float32),
                   jax.ShapeDtypeStruct((n, 1), jnp.float32),
                   jax.ShapeDtypeStruct((n, 1), jnp.float32)],
    )(num0, num0, den0, den0, h0, gat_b0.reshape(1, H), ln_g.reshape(1, H),
      ln_b.reshape(1, H), res_Wm0, res_bm0.reshape(1, MIDC), res_Wo0,
      res_bo0.reshape(1, H), res_g0.reshape(1, H), res_bb0.reshape(1, H),
      gat_W1, gat_as1.reshape(H, 1), gat_ad1.reshape(H, 1))

    num1, den1 = _sc_edge(src, dst, hw1,
                          jnp.concatenate([as1.reshape(-1), pad]),
                          jnp.concatenate([ad1.reshape(-1), pad]),
                          nchunks, nacc)

    y = pl.pallas_call(
        _tc_post1_body,
        grid=(grid,),
        in_specs=[nb, nb1, db, db1, _row_spec(), _full((1, H)), _full((1, H)),
                  _full((1, H)), _full((MIDC, H)), _full((1, MIDC)),
                  _full((H, MIDC)), _full((1, H)), _full((1, H)),
                  _full((1, H)), _full((H, H)), _full((1, H))],
        out_specs=_row_spec(),
        out_shape=jax.ShapeDtypeStruct((n, H), jnp.float32),
    )(num1, num1, den1, den1, h0, gat_b1.reshape(1, H), ln_g.reshape(1, H),
      ln_b.reshape(1, H), res_Wm1, res_bm1.reshape(1, MIDC), res_Wo1,
      res_bo1.reshape(1, H), res_g1.reshape(1, H), res_bb1.reshape(1, H),
      W_out, b_out.reshape(1, H))
    return y


# speedup vs baseline: 25.1251x; 25.1251x over previous
import functools

import jax
import jax.numpy as jnp
from jax import lax
from jax.experimental import pallas as pl
from jax.experimental.pallas import tpu as pltpu
from jax.experimental.pallas import tpu_sc as plsc

H = 128
MIDC = 512
NC, NS, LANES = 2, 16, 16
NW = NC * NS
CH = 128
TBL = 10016
ROW_BLK = 2000


def _dotc(a, b):
    return lax.dot_general(a, b, (((1,), (1,)), ((), ())),
                           preferred_element_type=jnp.float32)


def _layernorm(v, g, b):
    mu = jnp.mean(v, axis=-1, keepdims=True)
    var = jnp.mean((v - mu) ** 2, axis=-1, keepdims=True)
    return (v - mu) / jnp.sqrt(var + 1e-12) * g + b


def _gelu(v):
    return 0.5 * v * (1.0 + lax.erf(v * 0.7071067811865476))


def _tc_pre_body(x, wpre, bpre, w0, a_s, a_d, h0_o, hw_o, as_o, ad_o):
    h0 = jnp.maximum(_dotc(x[...], wpre[...]) + bpre[...], 0.0)
    hw = _dotc(h0, w0[...])
    h0_o[...] = h0
    hw_o[...] = hw
    as_o[...] = hw @ a_s[...]
    ad_o[...] = hw @ a_d[...]


def _post_block(num0, num1, den0, den1, prex, gatb, lng, lnb, wm, bm, wo, bo,
                rg, rb):
    agg = num0[0] + num1[0]
    den = den0[0] + den1[0] + 1e-16
    out = agg / den + gatb[...]
    h = _layernorm(prex[...] + _gelu(out), lng[...], lnb[...])
    mid = _gelu(_dotc(h, wm[...]) + bm[...])
    mid = _dotc(mid, wo[...]) + bo[...]
    return _layernorm(h + mid, rg[...], rb[...])


def _tc_post0_body(num0, num1, den0, den1, prex, gatb, lng, lnb, wm, bm, wo,
                   bo, rg, rb, w1, a_s, a_d, hw_o, as_o, ad_o):
    h2 = _post_block(num0, num1, den0, den1, prex, gatb, lng, lnb, wm, bm,
                     wo, bo, rg, rb)
    hw = _dotc(h2, w1[...])
    hw_o[...] = hw
    as_o[...] = hw @ a_s[...]
    ad_o[...] = hw @ a_d[...]


def _tc_post1_body(num0, num1, den0, den1, prex, gatb, lng, lnb, wm, bm, wo,
                   bo, rg, rb, wout, bout, y_o):
    h2 = _post_block(num0, num1, den0, den1, prex, gatb, lng, lnb, wm, bm,
                     wo, bo, rg, rb)
    y_o[...] = jnp.maximum(_dotc(h2, wout[...]) + bout[...], 0.0)


def _row_spec():
    return pl.BlockSpec((ROW_BLK, H), lambda i: (i, 0))


def _full(shape):
    return pl.BlockSpec(shape, lambda i: (0,) * len(shape))


def _sc_edge_body(nchunks, nacc, src_h, dst_h, hw_h, as_h, ad_h,
                  num_h, den_h, as_v, ad_v, srcv, dstv, rows, wbuf,
                  out_acc, den_acc, sem):
    cid = lax.axis_index("c")
    sid = lax.axis_index("s")
    wid = cid * NS + sid
    rows_per_tile = nacc // NS
    blocks_per_tile = rows_per_tile // CH

    pltpu.sync_copy(as_h, as_v)
    pltpu.sync_copy(ad_h, ad_v)

    z = jnp.zeros((LANES,), jnp.float32)

    def _zero_rows(j, _):
        for c in range(H // LANES):
            rows[j, pl.ds(c * LANES, LANES)] = z
        return 0

    lax.fori_loop(0, CH, _zero_rows, 0)
    for c in range(CH // LANES):
        wbuf[0, pl.ds(c * LANES, LANES)] = z
    for q in range(blocks_per_tile):
        r0 = sid * rows_per_tile + q * CH
        pltpu.sync_copy(rows, out_acc.at[pl.ds(r0, CH)])
        pltpu.sync_copy(wbuf.at[0], den_acc.at[pl.ds(r0, CH)])
    plsc.subcore_barrier()

    def _chunk(q, _):
        base = (wid * nchunks + q) * CH
        pltpu.sync_copy(src_h.at[pl.ds(base, CH)], srcv.at[0])
        pltpu.sync_copy(dst_h.at[pl.ds(base, CH)], dstv.at[0])
        cp = pltpu.async_copy(hw_h.at[srcv.at[0]], rows, sem)
        for i in range(CH // LANES):
            sv = srcv[0, pl.ds(i * LANES, LANES)]
            dv = dstv[0, pl.ds(i * LANES, LANES)]
            s = plsc.load_gather(as_v, [sv]) + plsc.load_gather(ad_v, [dv])
            e = jnp.where(s >= 0.0, s, 0.2 * s)
            wbuf[0, pl.ds(i * LANES, LANES)] = jnp.exp(e)
        cp.wait()

        def _scale(jg, _):
            wv = wbuf[0, pl.ds(jg * LANES, LANES)]
            for jj in range(LANES):
                j = jg * LANES + jj
                wj = wv[jj]
                for c in range(H // LANES):
                    sl = pl.ds(c * LANES, LANES)
                    rows[j, sl] = rows[j, sl] * wj
            return 0

        lax.fori_loop(0, CH // LANES, _scale, 0)
        pltpu.sync_copy(rows, out_acc.at[dstv.at[0]], add=True)
        pltpu.sync_copy(wbuf.at[0], den_acc.at[dstv.at[0]], add=True)
        return 0

    lax.fori_loop(0, nchunks, _chunk, 0)
    plsc.subcore_barrier()

    for q in range(blocks_per_tile):
        r0 = sid * rows_per_tile + q * CH
        pltpu.sync_copy(out_acc.at[pl.ds(r0, CH)],
                        num_h.at[cid].at[pl.ds(r0, CH)])
        pltpu.sync_copy(den_acc.at[pl.ds(r0, CH)],
                        den_h.at[cid].at[pl.ds(r0, CH)])


def _sc_edge(src, dst, hw, as_pad, ad_pad, nchunks, nacc):
    mesh = plsc.VectorSubcoreMesh(core_axis_name="c", subcore_axis_name="s",
                                  num_cores=NC, num_subcores=NS)
    fn = pl.kernel(
        functools.partial(_sc_edge_body, nchunks, nacc),
        out_type=(jax.ShapeDtypeStruct((NC, nacc, H), jnp.float32),
                  jax.ShapeDtypeStruct((NC, nacc), jnp.float32)),
        mesh=mesh,
        compiler_params=pltpu.CompilerParams(needs_layout_passes=False),
        scratch_types=(
            pltpu.VMEM((TBL,), jnp.float32),
            pltpu.VMEM((TBL,), jnp.float32),
            pltpu.VMEM((1, CH), jnp.int32),
            pltpu.VMEM((1, CH), jnp.int32),
            pltpu.VMEM((CH, H), jnp.float32),
            pltpu.VMEM((1, CH), jnp.float32),
            pltpu.VMEM_SHARED((nacc, H), jnp.float32),
            pltpu.VMEM_SHARED((nacc,), jnp.float32),
            pltpu.SemaphoreType.DMA,
        ),
    )
    return fn(src, dst, hw, as_pad, ad_pad)


def kernel(x, edge_index, W_pre, b_pre, gat_W0, gat_as0, gat_ad0, gat_b0,
           res_Wm0, res_bm0, res_Wo0, res_bo0, res_g0, res_bb0, gat_W1,
           gat_as1, gat_ad1, gat_b1, res_Wm1, res_bm1, res_Wo1, res_bo1,
           res_g1, res_bb1, ln_g, ln_b, W_out, b_out):
    n = x.shape[0]
    e = edge_index.shape[1]
    et = e + n
    nchunks = -(-et // (NW * CH))
    ep = NW * nchunks * CH
    nacc = -(-(n + LANES) // (NS * CH)) * NS * CH

    loops = jnp.arange(n, dtype=edge_index.dtype)
    src = jnp.concatenate([edge_index[0], loops,
                           jnp.zeros((ep - et,), edge_index.dtype)])
    dst = jnp.concatenate([edge_index[1], loops,
                           jnp.full((ep - et,), n, edge_index.dtype)])

    grid = n // ROW_BLK
    vec = pl.BlockSpec((H, 1), lambda i: (0, 0))
    row1 = pl.BlockSpec((ROW_BLK, 1), lambda i: (i, 0))
    nb = pl.BlockSpec((1, ROW_BLK, H), lambda i: (0, i, 0))
    nb1 = pl.BlockSpec((1, ROW_BLK, H), lambda i: (1, i, 0))
    db = pl.BlockSpec((1, ROW_BLK, 1), lambda i: (0, i, 0))
    db1 = pl.BlockSpec((1, ROW_BLK, 1), lambda i: (1, i, 0))

    h0, hw0, as0, ad0 = pl.pallas_call(
        _tc_pre_body,
        grid=(grid,),
        in_specs=[_row_spec(), _full((H, H)), _full((1, H)), _full((H, H)),
                  vec, vec],
        out_specs=[_row_spec(), _row_spec(), row1, row1],
        out_shape=[jax.ShapeDtypeStruct((n, H), jnp.float32),
                   jax.ShapeDtypeStruct((n, H), jnp.float32),
                   jax.ShapeDtypeStruct((n, 1), jnp.float32),
                   jax.ShapeDtypeStruct((n, 1), jnp.float32)],
    )(x, W_pre, b_pre.reshape(1, H), gat_W0, gat_as0.reshape(H, 1),
      gat_ad0.reshape(H, 1))

    pad = jnp.zeros((TBL - n,), jnp.float32)
    num0, den0 = _sc_edge(src, dst, hw0,
                          jnp.concatenate([as0.reshape(-1), pad]),
                          jnp.concatenate([ad0.reshape(-1), pad]),
                          nchunks, nacc)

    den0r = den0.reshape(NC, nacc, 1)
    hw1, as1, ad1 = pl.pallas_call(
        _tc_post0_body,
        grid=(grid,),
        in_specs=[nb, nb1, db, db1, _row_spec(), _full((1, H)), _full((1, H)),
                  _full((1, H)), _full((MIDC, H)), _full((1, MIDC)),
                  _full((H, MIDC)), _full((1, H)), _full((1, H)),
                  _full((1, H)), _full((H, H)), vec, vec],
        out_specs=[_row_spec(), row1, row1],
        out_shape=[jax.ShapeDtypeStruct((n, H), jnp.float32),
                   jax.ShapeDtypeStruct((n, 1), jnp.float32),
                   jax.ShapeDtypeStruct((n, 1), jnp.float32)],
    )(num0, num0, den0r, den0r, h0, gat_b0.reshape(1, H), ln_g.reshape(1, H),
      ln_b.reshape(1, H), res_Wm0, res_bm0.reshape(1, MIDC), res_Wo0,
      res_bo0.reshape(1, H), res_g0.reshape(1, H), res_bb0.reshape(1, H),
      gat_W1, gat_as1.reshape(H, 1), gat_ad1.reshape(H, 1))

    num1, den1 = _sc_edge(src, dst, hw1,
                          jnp.concatenate([as1.reshape(-1), pad]),
                          jnp.concatenate([ad1.reshape(-1), pad]),
                          nchunks, nacc)

    den1r = den1.reshape(NC, nacc, 1)
    y = pl.pallas_call(
        _tc_post1_body,
        grid=(grid,),
        in_specs=[nb, nb1, db, db1, _row_spec(), _full((1, H)), _full((1, H)),
                  _full((1, H)), _full((MIDC, H)), _full((1, MIDC)),
                  _full((H, MIDC)), _full((1, H)), _full((1, H)),
                  _full((1, H)), _full((H, H)), _full((1, H))],
        out_specs=_row_spec(),
        out_shape=jax.ShapeDtypeStruct((n, H), jnp.float32),
    )(num1, num1, den1r, den1r, h0, gat_b1.reshape(1, H), ln_g.reshape(1, H),
      ln_b.reshape(1, H), res_Wm1, res_bm1.reshape(1, MIDC), res_Wo1,
      res_bo1.reshape(1, H), res_g1.reshape(1, H), res_bb1.reshape(1, H),
      W_out, b_out.reshape(1, H))
    return y
